# baseline (speedup 1.0000x reference)
import functools

import jax
import jax.numpy as jnp
from jax import lax
from jax.experimental import pallas as pl
from jax.experimental.pallas import tpu as pltpu
from jax.experimental.pallas import tpu_sc as plsc

_NC = 2
_NS = 16
_NW = _NC * _NS
_D = 128
_DEG = 32
_G = 500


def _worker_id():
    return lax.axis_index("s") * _NC + lax.axis_index("c")


@functools.lru_cache(maxsize=None)
def _sc_gather(B):
    rows_per_worker = B // _NW
    n_chunk = rows_per_worker // 64

    def body(table_hbm, idx_hbm, out_hbm, idx_v, rows_v, sem):
        w = _worker_id()
        pltpu.sync_copy(idx_hbm.at[w], idx_v)
        base = w * rows_per_worker

        def chunk(j, carry):
            pltpu.async_copy(table_hbm.at[idx_v.at[j]], rows_v, sem).wait()
            pltpu.sync_copy(rows_v, out_hbm.at[pl.ds(base + j * 64, 64)])
            return carry

        lax.fori_loop(0, n_chunk, chunk, 0)

    return pl.kernel(
        body,
        out_type=jax.ShapeDtypeStruct((B, _D), jnp.float32),
        mesh=plsc.VectorSubcoreMesh(core_axis_name="c", subcore_axis_name="s"),
        scratch_types=[
            pltpu.VMEM((n_chunk, 64), jnp.int32),
            pltpu.VMEM((64, _D), jnp.float32),
            pltpu.SemaphoreType.DMA,
        ],
    )


@functools.lru_cache(maxsize=None)
def _sc_gather_sum(B, C, NBUF, FRAC0):
    rows_c0 = (B * FRAC0 // 16) // (_NS * 32) * 32
    rows_c1 = B // _NS - rows_c0
    assert rows_c1 >= 0 and rows_c1 % 32 == 0
    ng0, ng1 = rows_c0 // C, rows_c1 // C
    assert ng0 % NBUF == 0 and ng1 % NBUF == 0
    n_dma = (C * _DEG) // 128
    ir0, ir1 = ng0 * n_dma, ng1 * n_dma

    def body(table_hbm, idx_hbm, out_hbm, idx_v, rows_v, acc_v, *sems):
        gsem = sems[:NBUF]
        osem = sems[NBUF:]
        sid = lax.axis_index("s")
        cid = lax.axis_index("c")

        def fire(g, b):
            for t in range(n_dma):
                pltpu.async_copy(
                    table_hbm.at[idx_v.at[g * n_dma + t]],
                    rows_v.at[b].at[pl.ds(t * 128, 128)],
                    gsem[b])

        def drain(b):
            for t in range(n_dma):
                pltpu.make_async_copy(
                    table_hbm.at[idx_v.at[t]],
                    rows_v.at[b].at[pl.ds(t * 128, 128)],
                    gsem[b]).wait()

        def run(n_groups, idx_base, idx_len, out_base):
            pltpu.sync_copy(idx_hbm.at[pl.ds(idx_base, idx_len)],
                            idx_v.at[pl.ds(0, idx_len)])
            for b in range(NBUF - 1):
                fire(b, b)

            def ring(i, carry):
                for b in range(NBUF):
                    g = NBUF * i + b
                    nxt = g + NBUF - 1

                    @pl.when(nxt < n_groups)
                    def _():
                        fire(nxt, (b + NBUF - 1) % NBUF)

                    drain(b)

                    @pl.when(g >= NBUF)
                    def _():
                        pltpu.make_async_copy(
                            acc_v.at[b], out_hbm.at[pl.ds(out_base, C)],
                            osem[b]).wait()

                    def reduce_one(c, carry2):
                        for s in range(_D // 16):
                            vals = [rows_v[b, c * _DEG + k, pl.ds(s * 16, 16)]
                                    for k in range(_DEG)]
                            while len(vals) > 1:
                                nxt2 = [vals[j] + vals[j + 1]
                                        for j in range(0, len(vals) - 1, 2)]
                                if len(vals) % 2:
                                    nxt2.append(vals[-1])
                                vals = nxt2
                            acc_v[b, c, pl.ds(s * 16, 16)] = vals[0]
                        return carry2

                    lax.fori_loop(0, C, reduce_one, 0)
                    pltpu.async_copy(
                        acc_v.at[b], out_hbm.at[pl.ds(out_base + g * C, C)],
                        osem[b])
                return carry

            lax.fori_loop(0, n_groups // NBUF, ring, 0)
            for b in range(NBUF):
                pltpu.make_async_copy(
                    acc_v.at[b], out_hbm.at[pl.ds(out_base, C)],
                    osem[b]).wait()

        @pl.when(cid == 0)
        def _():
            run(ng0, sid * ir0, ir0, sid * rows_c0)

        if ng1 > 0:
            @pl.when(cid == 1)
            def _():
                run(ng1, _NS * ir0 + sid * ir1, ir1,
                    _NS * rows_c0 + sid * rows_c1)

    return pl.kernel(
        body,
        out_type=jax.ShapeDtypeStruct((B, _D), jnp.float32),
        mesh=plsc.VectorSubcoreMesh(core_axis_name="c", subcore_axis_name="s"),
        scratch_types=[
            pltpu.VMEM((ir0, 128), jnp.int32),
            pltpu.VMEM((NBUF, C * _DEG, _D), jnp.float32),
            pltpu.VMEM((NBUF, C, _D), jnp.float32),
        ] + [pltpu.SemaphoreType.DMA] * (2 * NBUF),
    )


def _k1_body(x_ref, Wn_ref, Ws_ref, bs_ref, y_ref, s_ref):
    x = x_ref[...]
    y_ref[0] = jnp.dot(x, Wn_ref[0], preferred_element_type=jnp.float32)
    s_ref[0] = (jnp.dot(x, Ws_ref[0], preferred_element_type=jnp.float32)
                + bs_ref[0, 0])


def _k2_body(s0_ref, g_ref, bn0_ref, Ws1_ref, bs1_ref, Wn1_ref,
             s1_ref, y1_ref):
    ha = jnp.maximum(s0_ref[0], 0.0)
    hb = jnp.maximum(g_ref[0] * (1.0 / _DEG) + bn0_ref[0, 0], 0.0)
    Ws1 = Ws1_ref[0]
    Wn1 = Wn1_ref[0]
    s1_ref[0] = (jnp.dot(ha, Ws1[:_D], preferred_element_type=jnp.float32)
                 + jnp.dot(hb, Ws1[_D:], preferred_element_type=jnp.float32)
                 + bs1_ref[0, 0])
    y1_ref[0] = (jnp.dot(ha, Wn1[:_D], preferred_element_type=jnp.float32)
                 + jnp.dot(hb, Wn1[_D:], preferred_element_type=jnp.float32))


def _k3_body(s1_ref, g_ref, bn1_ref, hid_ref, pool_ref):
    inv = 1.0 / _DEG
    a = jnp.maximum(s1_ref[0, 0], 0.0)
    b = jnp.maximum(g_ref[0, 0] * inv + bn1_ref[0, 0], 0.0)
    c = jnp.maximum(s1_ref[1, 0], 0.0)
    d = jnp.maximum(g_ref[1, 0] * inv + bn1_ref[1, 0], 0.0)
    hid = jnp.concatenate([a, b, c, d], axis=1)
    hid_ref[0] = hid
    pool_ref[0, 0] = jnp.max(hid, axis=0)


def kernel(fw_adjs, bw_adjs, features, emb,
           fw_Ws0, fw_bs0, fw_Wn0, fw_bn0, fw_Ws1, fw_bs1, fw_Wn1, fw_bn1,
           bw_Ws0, bw_bs0, bw_Wn0, bw_bn0, bw_Ws1, bw_bs1, bw_Wn1, bw_bn1):
    N = fw_adjs.shape[0]
    NP = -(-N // 512) * 512
    B = 2 * NP
    NG = N // _G

    feat_pad = jnp.concatenate(
        [features.astype(jnp.int32), jnp.zeros((NP - N,), jnp.int32)])
    pad = jnp.zeros((NP - N, _DEG), jnp.int32)
    fw_i = jnp.concatenate([fw_adjs.astype(jnp.int32), pad], axis=0)
    bw_i = jnp.concatenate([bw_adjs.astype(jnp.int32) + N, pad], axis=0)
    hop_idx = jnp.concatenate(
        [fw_i.reshape(-1), bw_i.reshape(-1)]).reshape(-1, 128)

    Wn0_s = jnp.stack([fw_Wn0, bw_Wn0])
    Ws0_s = jnp.stack([fw_Ws0, bw_Ws0])
    bs0_s = jnp.stack([fw_bs0, bw_bs0]).reshape(2, 1, _D)
    bn0_s = jnp.stack([fw_bn0, bw_bn0]).reshape(2, 1, _D)
    Ws1_s = jnp.stack([fw_Ws1, bw_Ws1])
    Wn1_s = jnp.stack([fw_Wn1, bw_Wn1])
    bs1_s = jnp.stack([fw_bs1, bw_bs1]).reshape(2, 1, _D)
    bn1_s = jnp.stack([fw_bn1, bw_bn1]).reshape(2, 1, _D)

    x = _sc_gather(NP)(emb, feat_pad.reshape(_NW, -1, 64))[:N]

    RB = 2000
    grid = (2, N // RB)
    w_spec = pl.BlockSpec((1, _D, _D), lambda d, i: (d, 0, 0))
    b_spec = pl.BlockSpec((1, 1, _D), lambda d, i: (d, 0, 0))
    r_spec = pl.BlockSpec((1, RB, _D), lambda d, i: (d, i, 0))
    y0, s0 = pl.pallas_call(
        _k1_body,
        grid=grid,
        in_specs=[pl.BlockSpec((RB, _D), lambda d, i: (i, 0)),
                  w_spec, w_spec, b_spec],
        out_specs=[r_spec, r_spec],
        out_shape=[jax.ShapeDtypeStruct((2, N, _D), jnp.float32),
                   jax.ShapeDtypeStruct((2, N, _D), jnp.float32)],
    )(x, Wn0_s, Ws0_s, bs0_s)

    g0 = _sc_gather_sum(B, 8, 2, 16)(y0.reshape(2 * N, _D), hop_idx)
    g0 = g0.reshape(2, NP, _D)[:, :N]

    w2_spec = pl.BlockSpec((1, 2 * _D, _D), lambda d, i: (d, 0, 0))
    s1, y1 = pl.pallas_call(
        _k2_body,
        grid=grid,
        in_specs=[r_spec, r_spec, b_spec, w2_spec, b_spec, w2_spec],
        out_specs=[r_spec, r_spec],
        out_shape=[jax.ShapeDtypeStruct((2, N, _D), jnp.float32),
                   jax.ShapeDtypeStruct((2, N, _D), jnp.float32)],
    )(s0, g0, bn0_s, Ws1_s, bs1_s, Wn1_s)

    g1 = _sc_gather_sum(B, 8, 2, 16)(y1.reshape(2 * N, _D), hop_idx)
    g1 = g1.reshape(2, NP, _D)[:, :N]

    pair_spec = pl.BlockSpec((2, 1, _G, _D), lambda g: (0, g, 0, 0))
    hidden, pooled = pl.pallas_call(
        _k3_body,
        grid=(NG,),
        in_specs=[pair_spec, pair_spec,
                  pl.BlockSpec((2, 1, _D), lambda g: (0, 0, 0))],
        out_specs=[pl.BlockSpec((1, _G, 4 * _D), lambda g: (g, 0, 0)),
                   pl.BlockSpec((1, 1, 4 * _D), lambda g: (g, 0, 0))],
        out_shape=[jax.ShapeDtypeStruct((NG, _G, 4 * _D), jnp.float32),
                   jax.ShapeDtypeStruct((NG, 1, 4 * _D), jnp.float32)],
    )(s1.reshape(2, NG, _G, _D), g1.reshape(2, NG, _G, _D), bn1_s)

    graph_embedding = pooled.reshape(NG, 4 * _D)
    return hidden, (graph_embedding, graph_embedding)


# speedup vs baseline: 1.9940x; 1.9940x over previous
import functools

import jax
import jax.numpy as jnp
import numpy as np
from jax import lax
from jax.experimental import pallas as pl
from jax.experimental.pallas import tpu as pltpu
from jax.experimental.pallas import tpu_sc as plsc

_NC = 2
_NS = 16
_NW = _NC * _NS
_D = 128
_DEG = 32
_G = 500


def _worker_id():
    return lax.axis_index("s") * _NC + lax.axis_index("c")


@functools.lru_cache(maxsize=None)
def _sc_gather(B):
    rows_per_worker = B // _NW
    n_chunk = rows_per_worker // 64

    def body(table_hbm, idx_hbm, out_hbm, idx_v, rows_v, sem):
        w = _worker_id()
        pltpu.sync_copy(idx_hbm.at[w], idx_v)
        base = w * rows_per_worker

        def chunk(j, carry):
            pltpu.async_copy(table_hbm.at[idx_v.at[j]], rows_v, sem).wait()
            pltpu.sync_copy(rows_v, out_hbm.at[pl.ds(base + j * 64, 64)])
            return carry

        lax.fori_loop(0, n_chunk, chunk, 0)

    return pl.kernel(
        body,
        out_type=jax.ShapeDtypeStruct((B, _D), jnp.float32),
        mesh=plsc.VectorSubcoreMesh(core_axis_name="c", subcore_axis_name="s"),
        scratch_types=[
            pltpu.VMEM((n_chunk, 64), jnp.int32),
            pltpu.VMEM((64, _D), jnp.float32),
            pltpu.SemaphoreType.DMA,
        ],
    )


@functools.lru_cache(maxsize=None)
def _sc_gather_sum(B, C, NBUF, FRAC0):
    rows_c0 = (B * FRAC0 // 16) // (_NS * 32) * 32
    rows_c1 = B // _NS - rows_c0
    assert rows_c1 >= 0 and rows_c1 % 32 == 0
    ng0, ng1 = rows_c0 // C, rows_c1 // C
    assert ng0 % NBUF == 0 and ng1 % NBUF == 0
    n_dma = (C * _DEG) // 128
    ir0, ir1 = ng0 * n_dma, ng1 * n_dma

    def body(table_hbm, idx_hbm, out_hbm, idx_v, rows_v, acc_v, *sems):
        gsem = sems[:NBUF]
        osem = sems[NBUF:]
        sid = lax.axis_index("s")
        cid = lax.axis_index("c")

        def fire(g, b):
            for t in range(n_dma):
                pltpu.async_copy(
                    table_hbm.at[idx_v.at[g * n_dma + t]],
                    rows_v.at[b].at[pl.ds(t * 128, 128)],
                    gsem[b])

        def drain(b):
            for t in range(n_dma):
                pltpu.make_async_copy(
                    table_hbm.at[idx_v.at[t]],
                    rows_v.at[b].at[pl.ds(t * 128, 128)],
                    gsem[b]).wait()

        def _tree_sum(vals):
            while len(vals) > 1:
                nxt = [vals[j] + vals[j + 1]
                       for j in range(0, len(vals) - 1, 2)]
                if len(vals) % 2:
                    nxt.append(vals[-1])
                vals = nxt
            return vals[0]

        def run(n_groups, idx_base, idx_len, out_base):
            pltpu.sync_copy(idx_hbm.at[pl.ds(idx_base, idx_len)],
                            idx_v.at[pl.ds(0, idx_len)])
            for b in range(NBUF - 1):
                fire(b, b)

            def ring(i, carry):
                for b in range(NBUF):
                    g = NBUF * i + b
                    nxt = g + NBUF - 1

                    @pl.when(nxt < n_groups)
                    def _():
                        fire(nxt, (b + NBUF - 1) % NBUF)

                    drain(b)

                    @pl.when(g >= NBUF)
                    def _():
                        pltpu.make_async_copy(
                            acc_v.at[b], out_hbm.at[pl.ds(out_base, C)],
                            osem[b]).wait()

                    def reduce_one(c, carry2):
                        for s in range(4):
                            los, his = [], []
                            for k in range(_DEG):
                                w16 = rows_v[b, c * _DEG + k,
                                             pl.ds(s * 16, 16)]
                                los.append(lax.bitcast_convert_type(
                                    jnp.left_shift(w16, 16), jnp.float32))
                                his.append(lax.bitcast_convert_type(
                                    jnp.bitwise_and(
                                        w16, jnp.int32(-65536)),
                                    jnp.float32))
                            acc_v[b, c, pl.ds(s * 16, 16)] = _tree_sum(los)
                            acc_v[b, c, pl.ds(64 + s * 16, 16)] = \
                                _tree_sum(his)
                        return carry2

                    lax.fori_loop(0, C, reduce_one, 0)
                    pltpu.async_copy(
                        acc_v.at[b], out_hbm.at[pl.ds(out_base + g * C, C)],
                        osem[b])
                return carry

            lax.fori_loop(0, n_groups // NBUF, ring, 0)
            for b in range(NBUF):
                pltpu.make_async_copy(
                    acc_v.at[b], out_hbm.at[pl.ds(out_base, C)],
                    osem[b]).wait()

        @pl.when(cid == 0)
        def _():
            run(ng0, sid * ir0, ir0, sid * rows_c0)

        if ng1 > 0:
            @pl.when(cid == 1)
            def _():
                run(ng1, _NS * ir0 + sid * ir1, ir1,
                    _NS * rows_c0 + sid * rows_c1)

    return pl.kernel(
        body,
        out_type=jax.ShapeDtypeStruct((B, _D), jnp.float32),
        mesh=plsc.VectorSubcoreMesh(core_axis_name="c", subcore_axis_name="s"),
        compiler_params=pltpu.CompilerParams(use_tc_tiling_on_sc=False),
        scratch_types=[
            pltpu.VMEM((ir0, 128), jnp.int32),
            pltpu.VMEM((NBUF, C * _DEG, _D // 2), jnp.int32),
            pltpu.VMEM((NBUF, C, _D), jnp.float32),
        ] + [pltpu.SemaphoreType.DMA] * (2 * NBUF),
    )


def _pack_bf16_pairs(y):
    yb = y.astype(jnp.bfloat16)
    lo = jax.lax.bitcast_convert_type(yb[:, :64], jnp.uint16).astype(jnp.uint32)
    hi = jax.lax.bitcast_convert_type(yb[:, 64:], jnp.uint16).astype(jnp.uint32)
    return jax.lax.bitcast_convert_type(
        jnp.bitwise_or(jnp.left_shift(hi, 16), lo), jnp.int32)


def _k1_body(x_ref, Wn_ref, Ws_ref, bs_ref, y_ref, s_ref):
    x = x_ref[...]
    y_ref[0] = _pack_bf16_pairs(
        jnp.dot(x, Wn_ref[0], preferred_element_type=jnp.float32))
    s_ref[0] = (jnp.dot(x, Ws_ref[0], preferred_element_type=jnp.float32)
                + bs_ref[0, 0])


def _k2_body(s0_ref, g_ref, bn0_ref, Ws1_ref, bs1_ref, Wn1_ref,
             s1_ref, y1_ref):
    ha = jnp.maximum(s0_ref[0], 0.0)
    hb = jnp.maximum(g_ref[0] * (1.0 / _DEG) + bn0_ref[0, 0], 0.0)
    Ws1 = Ws1_ref[0]
    Wn1 = Wn1_ref[0]
    s1_ref[0] = (jnp.dot(ha, Ws1[:_D], preferred_element_type=jnp.float32)
                 + jnp.dot(hb, Ws1[_D:], preferred_element_type=jnp.float32)
                 + bs1_ref[0, 0])
    y1_ref[0] = _pack_bf16_pairs(
        jnp.dot(ha, Wn1[:_D], preferred_element_type=jnp.float32)
        + jnp.dot(hb, Wn1[_D:], preferred_element_type=jnp.float32))


def _k3_body(s1_ref, g_ref, bn1_ref, hid_ref, pool_ref):
    inv = 1.0 / _DEG
    a = jnp.maximum(s1_ref[0, 0], 0.0)
    b = jnp.maximum(g_ref[0, 0] * inv + bn1_ref[0, 0], 0.0)
    c = jnp.maximum(s1_ref[1, 0], 0.0)
    d = jnp.maximum(g_ref[1, 0] * inv + bn1_ref[1, 0], 0.0)
    hid = jnp.concatenate([a, b, c, d], axis=1)
    hid_ref[0] = hid
    pool_ref[0, 0] = jnp.max(hid, axis=0)


def kernel(fw_adjs, bw_adjs, features, emb,
           fw_Ws0, fw_bs0, fw_Wn0, fw_bn0, fw_Ws1, fw_bs1, fw_Wn1, fw_bn1,
           bw_Ws0, bw_bs0, bw_Wn0, bw_bn0, bw_Ws1, bw_bs1, bw_Wn1, bw_bn1):
    N = fw_adjs.shape[0]
    NP = -(-N // 512) * 512
    B = 2 * NP
    NG = N // _G

    feat_pad = jnp.concatenate(
        [features.astype(jnp.int32), jnp.zeros((NP - N,), jnp.int32)])
    pad = jnp.zeros((NP - N, _DEG), jnp.int32)
    fw_i = jnp.concatenate([fw_adjs.astype(jnp.int32), pad], axis=0)
    bw_i = jnp.concatenate([bw_adjs.astype(jnp.int32) + N, pad], axis=0)
    hop_idx = jnp.concatenate(
        [fw_i.reshape(-1), bw_i.reshape(-1)]).reshape(-1, 128)

    Wn0_s = jnp.stack([fw_Wn0, bw_Wn0])
    Ws0_s = jnp.stack([fw_Ws0, bw_Ws0])
    bs0_s = jnp.stack([fw_bs0, bw_bs0]).reshape(2, 1, _D)
    bn0_s = jnp.stack([fw_bn0, bw_bn0]).reshape(2, 1, _D)
    Ws1_s = jnp.stack([fw_Ws1, bw_Ws1])
    Wn1_s = jnp.stack([fw_Wn1, bw_Wn1])
    bs1_s = jnp.stack([fw_bs1, bw_bs1]).reshape(2, 1, _D)
    bn1_s = jnp.stack([fw_bn1, bw_bn1]).reshape(2, 1, _D)

    x = _sc_gather(NP)(emb, feat_pad.reshape(_NW, -1, 64))[:N]

    RB = 2000
    grid = (2, N // RB)
    w_spec = pl.BlockSpec((1, _D, _D), lambda d, i: (d, 0, 0))
    b_spec = pl.BlockSpec((1, 1, _D), lambda d, i: (d, 0, 0))
    r_spec = pl.BlockSpec((1, RB, _D), lambda d, i: (d, i, 0))
    p_spec = pl.BlockSpec((1, RB, _D // 2), lambda d, i: (d, i, 0))
    y0, s0 = pl.pallas_call(
        _k1_body,
        grid=grid,
        in_specs=[pl.BlockSpec((RB, _D), lambda d, i: (i, 0)),
                  w_spec, w_spec, b_spec],
        out_specs=[p_spec, r_spec],
        out_shape=[jax.ShapeDtypeStruct((2, N, _D // 2), jnp.int32),
                   jax.ShapeDtypeStruct((2, N, _D), jnp.float32)],
    )(x, Wn0_s, Ws0_s, bs0_s)

    g0 = _sc_gather_sum(B, 16, 2, 8)(y0.reshape(2 * N, _D // 2), hop_idx)
    g0 = g0.reshape(2, NP, _D)[:, :N]

    w2_spec = pl.BlockSpec((1, 2 * _D, _D), lambda d, i: (d, 0, 0))
    s1, y1 = pl.pallas_call(
        _k2_body,
        grid=grid,
        in_specs=[r_spec, r_spec, b_spec, w2_spec, b_spec, w2_spec],
        out_specs=[r_spec, p_spec],
        out_shape=[jax.ShapeDtypeStruct((2, N, _D), jnp.float32),
                   jax.ShapeDtypeStruct((2, N, _D // 2), jnp.int32)],
    )(s0, g0, bn0_s, Ws1_s, bs1_s, Wn1_s)

    g1 = _sc_gather_sum(B, 16, 2, 8)(y1.reshape(2 * N, _D // 2), hop_idx)
    g1 = g1.reshape(2, NP, _D)[:, :N]

    pair_spec = pl.BlockSpec((2, 1, _G, _D), lambda g: (0, g, 0, 0))
    hidden, pooled = pl.pallas_call(
        _k3_body,
        grid=(NG,),
        in_specs=[pair_spec, pair_spec,
                  pl.BlockSpec((2, 1, _D), lambda g: (0, 0, 0))],
        out_specs=[pl.BlockSpec((1, _G, 4 * _D), lambda g: (g, 0, 0)),
                   pl.BlockSpec((1, 1, 4 * _D), lambda g: (g, 0, 0))],
        out_shape=[jax.ShapeDtypeStruct((NG, _G, 4 * _D), jnp.float32),
                   jax.ShapeDtypeStruct((NG, 1, 4 * _D), jnp.float32)],
    )(s1.reshape(2, NG, _G, _D), g1.reshape(2, NG, _G, _D), bn1_s)

    graph_embedding = pooled.reshape(NG, 4 * _D)
    return hidden, (graph_embedding, graph_embedding)
